# Initial kernel scaffold; baseline (speedup 1.0000x reference)
import jax
import jax.numpy as jnp
from jax.experimental import pallas as pl


def kernel(x, edge_index, W_self0, W_neigh0, b0, W_self1, W_neigh1, b1):
    raise NotImplementedError("write your pallas kernel here")


# speedup vs baseline: 5.3498x; 5.3498x over previous
import functools

import jax
import jax.numpy as jnp
from jax import lax
from jax.experimental import pallas as pl
from jax.experimental.pallas import tpu as pltpu
from jax.experimental.pallas import tpu_sc as plsc

N_NODES = 10000
N_EDGES = 320000
D = 128

_NC = 2
_NS = 16
_NW = _NC * _NS
_EPW = N_EDGES // _NW
_K = 80
_NCHUNK = _EPW // _K
_ZROWS = 8
_SPAN = 632
_SPANL = N_NODES - (_NS - 1) * _SPAN


def _sc_body(with_deg, h_hbm, src_hbm, dst_hbm, agg_hbm, *rest):
    if with_deg:
        (deg0_hbm, deg1_hbm, sidx_v, didx_v, rows_v, zrows_v, ones_v, zed_v,
         agg_sh, deg_sh, sem) = rest
    else:
        sidx_v, didx_v, rows_v, zrows_v, agg_sh, sem = rest
    c = lax.axis_index("c")
    s = lax.axis_index("s")
    wid = c * _NS + s

    zvec = jnp.zeros((16,), jnp.float32)
    for r in range(_ZROWS):
        for j in range(D // 16):
            zrows_v[r, pl.ds(16 * j, 16)] = zvec
    row0 = pl.multiple_of(s * _SPAN, 8)
    span = jnp.where(s < _NS - 1, _SPAN, _SPANL)

    def zero_rows(i, carry):
        pltpu.sync_copy(zrows_v,
                        agg_sh.at[pl.ds(pl.multiple_of(row0 + i * _ZROWS, 8),
                                        _ZROWS)])
        return carry

    lax.fori_loop(0, span // _ZROWS, zero_rows, 0)

    if with_deg:
        for j in range(_K // 16):
            ones_v[pl.ds(16 * j, 16)] = jnp.ones((16,), jnp.float32)

        @pl.when(s == 0)
        def _():
            def zfill(i, carry):
                zed_v[pl.ds(i * 16, 16)] = zvec
                return carry

            lax.fori_loop(0, N_NODES // 16, zfill, 0)
            pltpu.sync_copy(zed_v, deg_sh)

    plsc.subcore_barrier()

    base = wid * _EPW

    def chunk(i, carry):
        off = pl.multiple_of(base + i * _K, 8)
        pltpu.sync_copy(src_hbm.at[pl.ds(off, _K)], sidx_v)
        pltpu.sync_copy(dst_hbm.at[pl.ds(off, _K)], didx_v)
        pltpu.async_copy(h_hbm.at[sidx_v], rows_v, sem).wait()
        pltpu.sync_copy(rows_v, agg_sh.at[didx_v], add=True)
        if with_deg:
            pltpu.sync_copy(ones_v, deg_sh.at[didx_v], add=True)
        return carry

    lax.fori_loop(0, _NCHUNK, chunk, 0)
    plsc.subcore_barrier()

    @pl.when(s < _NS - 1)
    def _():
        pltpu.sync_copy(agg_sh.at[pl.ds(row0, _SPAN)],
                        agg_hbm.at[c, pl.ds(row0, _SPAN)])

    @pl.when(s == _NS - 1)
    def _():
        pltpu.sync_copy(agg_sh.at[pl.ds(row0, _SPANL)],
                        agg_hbm.at[c, pl.ds(row0, _SPANL)])

    if with_deg:
        @pl.when(jnp.logical_and(s == 0, c == 0))
        def _():
            pltpu.sync_copy(deg_sh, deg0_hbm)

        @pl.when(jnp.logical_and(s == 0, c == 1))
        def _():
            pltpu.sync_copy(deg_sh, deg1_hbm)


def _make_sc_agg(with_deg, interpret=False):
    out_type = [jax.ShapeDtypeStruct((_NC, N_NODES, D), jnp.float32)]
    scratch = [
        pltpu.VMEM((_K,), jnp.int32),
        pltpu.VMEM((_K,), jnp.int32),
        pltpu.VMEM((_K, D), jnp.float32),
        pltpu.VMEM((_ZROWS, D), jnp.float32),
    ]
    if with_deg:
        out_type += [jax.ShapeDtypeStruct((N_NODES,), jnp.float32),
                     jax.ShapeDtypeStruct((N_NODES,), jnp.float32)]
        scratch += [
            pltpu.VMEM((_K,), jnp.float32),
            pltpu.VMEM((N_NODES,), jnp.float32),
        ]
    scratch.append(pltpu.VMEM_SHARED((N_NODES, D), jnp.float32))
    if with_deg:
        scratch.append(pltpu.VMEM_SHARED((N_NODES,), jnp.float32))
    scratch.append(pltpu.SemaphoreType.DMA)
    mesh = plsc.VectorSubcoreMesh(core_axis_name="c", subcore_axis_name="s")
    return pl.kernel(
        functools.partial(_sc_body, with_deg),
        out_type=tuple(out_type) if with_deg else out_type[0],
        mesh=mesh,
        scratch_types=scratch,
        interpret=interpret,
        name="sage_sc_agg_deg" if with_deg else "sage_sc_agg",
    )


_R = 1000


def _tc_body(relu, h_ref, a0_ref, a1_ref, d0_ref, d1_ref, ws_ref, wn_ref, b_ref,
             o_ref):
    deg = jnp.maximum(d0_ref[...] + d1_ref[...], 1.0)
    neigh = (a0_ref[...] + a1_ref[...]) / deg
    y = jnp.dot(h_ref[...], ws_ref[...], preferred_element_type=jnp.float32)
    y += jnp.dot(neigh, wn_ref[...], preferred_element_type=jnp.float32)
    y += b_ref[...]
    if relu:
        y = jnp.maximum(y, 0.0)
    o_ref[...] = y


def _make_tc_layer(relu):
    blk = lambda i: (i, 0)
    full = lambda i: (0, 0)
    return pl.pallas_call(
        functools.partial(_tc_body, relu),
        grid=(N_NODES // _R,),
        in_specs=[
            pl.BlockSpec((_R, D), blk),
            pl.BlockSpec((_R, D), blk),
            pl.BlockSpec((_R, D), blk),
            pl.BlockSpec((_R, 1), blk),
            pl.BlockSpec((_R, 1), blk),
            pl.BlockSpec((D, D), full),
            pl.BlockSpec((D, D), full),
            pl.BlockSpec((1, D), full),
        ],
        out_specs=pl.BlockSpec((_R, D), blk),
        out_shape=jax.ShapeDtypeStruct((N_NODES, D), jnp.float32),
        name="sage_tc_layer",
    )


_sc_agg_deg = _make_sc_agg(with_deg=True)
_sc_agg = _make_sc_agg(with_deg=False)
_tc_layer_relu = _make_tc_layer(relu=True)
_tc_layer_lin = _make_tc_layer(relu=False)


def kernel(x, edge_index, W_self0, W_neigh0, b0, W_self1, W_neigh1, b1):
    src = edge_index[0].astype(jnp.int32)
    dst = edge_index[1].astype(jnp.int32)
    aggp, deg0, deg1 = _sc_agg_deg(x, src, dst)
    d0 = deg0.reshape(N_NODES, 1)
    d1 = deg1.reshape(N_NODES, 1)
    h = _tc_layer_relu(x, aggp[0], aggp[1], d0, d1, W_self0, W_neigh0,
                       b0.reshape(1, D))
    aggp2 = _sc_agg(h, src, dst)
    return _tc_layer_lin(h, aggp2[0], aggp2[1], d0, d1, W_self1, W_neigh1,
                         b1.reshape(1, D))


# speedup vs baseline: 9.9903x; 1.8674x over previous
import functools

import jax
import jax.numpy as jnp
from jax import lax
from jax.experimental import pallas as pl
from jax.experimental.pallas import tpu as pltpu
from jax.experimental.pallas import tpu_sc as plsc

N_NODES = 10000
N_EDGES = 320000
D = 128

_NC = 2
_NS = 16
_NW = _NC * _NS
_EPW = N_EDGES // _NW
_K = 80
_NCHUNK = _EPW // _K
_NBUF = 2
_NG = _NCHUNK // _NBUF
_NTAIL = _NCHUNK - _NG * _NBUF
_ZROWS = 8
_ZED = 1000
_SPAN = 632
_SPANL = N_NODES - (_NS - 1) * _SPAN


def _sc_body(with_deg, h_hbm, src_hbm, dst_hbm, agg_hbm, *rest):
    if with_deg:
        deg0_hbm, deg1_hbm = rest[0], rest[1]
        rest = rest[2:]
        (sidx1d, didx1d, sbuf, dbuf, rows_v, ones_v, zed_v, agg_sh,
         deg_sh) = rest[:9]
        sems = rest[9:]
        gsem, ssem, dsem = sems[:_NBUF], sems[_NBUF:2 * _NBUF], sems[2 * _NBUF:]
    else:
        (sidx1d, didx1d, sbuf, dbuf, rows_v, agg_sh) = rest[:6]
        sems = rest[6:]
        gsem, ssem = sems[:_NBUF], sems[_NBUF:]
    c = lax.axis_index("c")
    s = lax.axis_index("s")
    wid = c * _NS + s

    ebase = pl.multiple_of(wid * _EPW, 8)
    pltpu.sync_copy(src_hbm.at[pl.ds(ebase, _EPW)], sidx1d)
    pltpu.sync_copy(dst_hbm.at[pl.ds(ebase, _EPW)], didx1d)

    zvec = jnp.zeros((16,), jnp.float32)
    for r in range(_ZROWS):
        for j in range(D // 16):
            rows_v[0, r, pl.ds(16 * j, 16)] = zvec
    zstage = rows_v.at[0].at[pl.ds(0, _ZROWS)]
    row0 = pl.multiple_of(s * _SPAN, 8)
    span = jnp.where(s < _NS - 1, _SPAN, _SPANL)

    def zero_rows(i, carry):
        pltpu.sync_copy(zstage,
                        agg_sh.at[pl.ds(pl.multiple_of(row0 + i * _ZROWS, 8),
                                        _ZROWS)])
        return carry

    lax.fori_loop(0, span // _ZROWS, zero_rows, 0)

    if with_deg:
        for j in range(_K // 16 + 1):
            ones_v[pl.ds(16 * j, 16)] = jnp.ones((16,), jnp.float32)

        @pl.when(s == 0)
        def _():
            def zfill(i, carry):
                zed_v[pl.ds(i * 16, 16)] = zvec
                return carry

            lax.fori_loop(0, _ZED // 16, zfill, 0)

            def zdeg(i, carry):
                pltpu.sync_copy(zed_v,
                                deg_sh.at[pl.ds(pl.multiple_of(i * _ZED, 8),
                                                _ZED)])
                return carry

            lax.fori_loop(0, N_NODES // _ZED, zdeg, 0)

    plsc.subcore_barrier()

    def idx_stage(i, b):
        off = pl.multiple_of(i * _K, 16)
        for j in range(_K // 16):
            sbuf[b, pl.ds(16 * j, 16)] = sidx1d[pl.ds(off + 16 * j, 16)]
            dbuf[b, pl.ds(16 * j, 16)] = didx1d[pl.ds(off + 16 * j, 16)]

    def g_start(b):
        pltpu.async_copy(h_hbm.at[sbuf.at[b]], rows_v.at[b], gsem[b])

    def g_wait(b):
        pltpu.make_async_copy(h_hbm.at[sbuf.at[b]], rows_v.at[b],
                              gsem[b]).wait()

    def s_start(b):
        pltpu.async_copy(rows_v.at[b], agg_sh.at[dbuf.at[b]], ssem[b],
                         add=True)
        if with_deg:
            pltpu.async_copy(ones_v.at[pl.ds(0, _K)], deg_sh.at[dbuf.at[b]],
                             dsem[b], add=True)

    def s_wait(b):
        pltpu.make_async_copy(rows_v.at[b], agg_sh.at[dbuf.at[b]],
                              ssem[b]).wait()
        if with_deg:
            pltpu.make_async_copy(ones_v.at[pl.ds(0, _K)],
                                  deg_sh.at[dbuf.at[b]], dsem[b]).wait()

    for b in range(_NBUF):
        idx_stage(b, b)
        g_start(b)

    def group(g, carry):
        for b in range(_NBUF):
            g_wait(b)
            s_start(b)
        for b in range(_NBUF):
            i = g * _NBUF + b
            s_wait(b)

            @pl.when(g < _NG - 1)
            def _():
                idx_stage(i + _NBUF, b)
                g_start(b)
        return carry

    lax.fori_loop(0, _NG, group, 0)
    for j in range(_NTAIL):
        idx_stage(_NG * _NBUF + j, 0)
        g_start(0)
        g_wait(0)
        s_start(0)
        s_wait(0)
    plsc.subcore_barrier()

    @pl.when(s < _NS - 1)
    def _():
        pltpu.sync_copy(agg_sh.at[pl.ds(row0, _SPAN)],
                        agg_hbm.at[c, pl.ds(row0, _SPAN)])

    @pl.when(s == _NS - 1)
    def _():
        pltpu.sync_copy(agg_sh.at[pl.ds(row0, _SPANL)],
                        agg_hbm.at[c, pl.ds(row0, _SPANL)])

    if with_deg:
        @pl.when(jnp.logical_and(s == 0, c == 0))
        def _():
            pltpu.sync_copy(deg_sh, deg0_hbm)

        @pl.when(jnp.logical_and(s == 0, c == 1))
        def _():
            pltpu.sync_copy(deg_sh, deg1_hbm)


def _make_sc_agg(with_deg, interpret=False):
    out_type = [jax.ShapeDtypeStruct((_NC, N_NODES, D), jnp.float32)]
    scratch = [
        pltpu.VMEM((_EPW,), jnp.int32),
        pltpu.VMEM((_EPW,), jnp.int32),
        pltpu.VMEM((_NBUF, _K), jnp.int32),
        pltpu.VMEM((_NBUF, _K), jnp.int32),
        pltpu.VMEM((_NBUF, _K, D), jnp.float32),
    ]
    if with_deg:
        out_type += [jax.ShapeDtypeStruct((N_NODES,), jnp.float32),
                     jax.ShapeDtypeStruct((N_NODES,), jnp.float32)]
        scratch += [
            pltpu.VMEM((_K + 16,), jnp.float32),
            pltpu.VMEM((_ZED,), jnp.float32),
        ]
    scratch.append(pltpu.VMEM_SHARED((N_NODES, D), jnp.float32))
    if with_deg:
        scratch.append(pltpu.VMEM_SHARED((N_NODES,), jnp.float32))
    nsem = 3 * _NBUF if with_deg else 2 * _NBUF
    scratch += [pltpu.SemaphoreType.DMA] * nsem
    mesh = plsc.VectorSubcoreMesh(core_axis_name="c", subcore_axis_name="s")
    return pl.kernel(
        functools.partial(_sc_body, with_deg),
        out_type=tuple(out_type) if with_deg else out_type[0],
        mesh=mesh,
        scratch_types=scratch,
        interpret=interpret,
        name="sage_sc_agg_deg" if with_deg else "sage_sc_agg",
    )


_R = 1000


def _tc_body(relu, h_ref, a0_ref, a1_ref, d0_ref, d1_ref, ws_ref, wn_ref, b_ref,
             o_ref):
    deg = jnp.maximum(d0_ref[...] + d1_ref[...], 1.0)
    neigh = (a0_ref[...] + a1_ref[...]) / deg
    y = jnp.dot(h_ref[...], ws_ref[...], preferred_element_type=jnp.float32)
    y += jnp.dot(neigh, wn_ref[...], preferred_element_type=jnp.float32)
    y += b_ref[...]
    if relu:
        y = jnp.maximum(y, 0.0)
    o_ref[...] = y


def _make_tc_layer(relu):
    blk = lambda i: (i, 0)
    full = lambda i: (0, 0)
    return pl.pallas_call(
        functools.partial(_tc_body, relu),
        grid=(N_NODES // _R,),
        in_specs=[
            pl.BlockSpec((_R, D), blk),
            pl.BlockSpec((_R, D), blk),
            pl.BlockSpec((_R, D), blk),
            pl.BlockSpec((_R, 1), blk),
            pl.BlockSpec((_R, 1), blk),
            pl.BlockSpec((D, D), full),
            pl.BlockSpec((D, D), full),
            pl.BlockSpec((1, D), full),
        ],
        out_specs=pl.BlockSpec((_R, D), blk),
        out_shape=jax.ShapeDtypeStruct((N_NODES, D), jnp.float32),
        name="sage_tc_layer",
    )


_sc_agg_deg = _make_sc_agg(with_deg=True)
_sc_agg = _make_sc_agg(with_deg=False)
_tc_layer_relu = _make_tc_layer(relu=True)
_tc_layer_lin = _make_tc_layer(relu=False)


def kernel(x, edge_index, W_self0, W_neigh0, b0, W_self1, W_neigh1, b1):
    src = edge_index[0].astype(jnp.int32)
    dst = edge_index[1].astype(jnp.int32)
    aggp, deg0, deg1 = _sc_agg_deg(x, src, dst)
    d0 = deg0.reshape(N_NODES, 1)
    d1 = deg1.reshape(N_NODES, 1)
    h = _tc_layer_relu(x, aggp[0], aggp[1], d0, d1, W_self0, W_neigh0,
                       b0.reshape(1, D))
    aggp2 = _sc_agg(h, src, dst)
    return _tc_layer_lin(h, aggp2[0], aggp2[1], d0, d1, W_self1, W_neigh1,
                         b1.reshape(1, D))


# speedup vs baseline: 12.2931x; 1.2305x over previous
import functools

import jax
import jax.numpy as jnp
from jax import lax
from jax.experimental import pallas as pl
from jax.experimental.pallas import tpu as pltpu
from jax.experimental.pallas import tpu_sc as plsc

N_NODES = 10000
N_EDGES = 320000
D = 128

_NC = 2
_NS = 16
_NW = _NC * _NS
_EPW = N_EDGES // _NW
_K = 80
_NCHUNK = _EPW // _K
_NBUF = 4
_NG = _NCHUNK // _NBUF
_NTAIL = _NCHUNK - _NG * _NBUF
_ZROWS = 8
_ZED = 1000
_SPAN = 632
_SPANL = N_NODES - (_NS - 1) * _SPAN


def _sc_body(with_deg, h_hbm, src_hbm, dst_hbm, agg_hbm, *rest):
    if with_deg:
        deg0_hbm, deg1_hbm = rest[0], rest[1]
        rest = rest[2:]
        (sbuf, dbuf, rows_v, ones_v, zed_v, agg_sh, deg_sh) = rest[:7]
        sems = rest[7:]
    else:
        (sbuf, dbuf, rows_v, agg_sh) = rest[:4]
        sems = rest[4:]
    gsem, ssem = sems[:_NBUF], sems[_NBUF:2 * _NBUF]
    issem, idsem = sems[2 * _NBUF:3 * _NBUF], sems[3 * _NBUF:4 * _NBUF]
    if with_deg:
        dsem = sems[4 * _NBUF:]
    c = lax.axis_index("c")
    s = lax.axis_index("s")
    wid = c * _NS + s

    def _ioff(i):
        return pl.multiple_of(wid * _EPW + i * _K, 16)

    def isb_start(i, b):
        pltpu.async_copy(src_hbm.at[pl.ds(_ioff(i), _K)], sbuf.at[b], issem[b])

    def isb_wait(i, b):
        pltpu.make_async_copy(src_hbm.at[pl.ds(_ioff(i), _K)], sbuf.at[b],
                              issem[b]).wait()

    def idb_start(i, b):
        pltpu.async_copy(dst_hbm.at[pl.ds(_ioff(i), _K)], dbuf.at[b], idsem[b])

    def idb_wait(i, b):
        pltpu.make_async_copy(dst_hbm.at[pl.ds(_ioff(i), _K)], dbuf.at[b],
                              idsem[b]).wait()

    for b in range(_NBUF):
        isb_start(b, b)
        idb_start(b, b)

    zvec = jnp.zeros((16,), jnp.float32)
    for r in range(_ZROWS):
        for j in range(D // 16):
            rows_v[0, r, pl.ds(16 * j, 16)] = zvec
    zstage = rows_v.at[0].at[pl.ds(0, _ZROWS)]
    row0 = pl.multiple_of(s * _SPAN, 8)
    span = jnp.where(s < _NS - 1, _SPAN, _SPANL)

    def zero_rows(i, carry):
        pltpu.sync_copy(zstage,
                        agg_sh.at[pl.ds(pl.multiple_of(row0 + i * _ZROWS, 8),
                                        _ZROWS)])
        return carry

    lax.fori_loop(0, span // _ZROWS, zero_rows, 0)

    if with_deg:
        for j in range(_K // 16 + 1):
            ones_v[pl.ds(16 * j, 16)] = jnp.ones((16,), jnp.float32)

        @pl.when(s == 0)
        def _():
            def zfill(i, carry):
                zed_v[pl.ds(i * 16, 16)] = zvec
                return carry

            lax.fori_loop(0, _ZED // 16, zfill, 0)

            def zdeg(i, carry):
                pltpu.sync_copy(zed_v,
                                deg_sh.at[pl.ds(pl.multiple_of(i * _ZED, 8),
                                                _ZED)])
                return carry

            lax.fori_loop(0, N_NODES // _ZED, zdeg, 0)

    plsc.subcore_barrier()

    def g_start(b):
        pltpu.async_copy(h_hbm.at[sbuf.at[b]], rows_v.at[b], gsem[b])

    def g_wait(b):
        pltpu.make_async_copy(h_hbm.at[sbuf.at[b]], rows_v.at[b],
                              gsem[b]).wait()

    def s_start(b):
        pltpu.async_copy(rows_v.at[b], agg_sh.at[dbuf.at[b]], ssem[b],
                         add=True)
        if with_deg:
            pltpu.async_copy(ones_v.at[pl.ds(0, _K)], deg_sh.at[dbuf.at[b]],
                             dsem[b], add=True)

    def s_wait(b):
        pltpu.make_async_copy(rows_v.at[b], agg_sh.at[dbuf.at[b]],
                              ssem[b]).wait()
        if with_deg:
            pltpu.make_async_copy(ones_v.at[pl.ds(0, _K)],
                                  deg_sh.at[dbuf.at[b]], dsem[b]).wait()

    for b in range(_NBUF):
        isb_wait(b, b)
        g_start(b)

    def group(g, carry):
        for b in range(_NBUF):
            i = g * _NBUF + b
            g_wait(b)

            @pl.when(i + _NBUF < _NCHUNK)
            def _():
                isb_start(i + _NBUF, b)

            idb_wait(i, b)
            s_start(b)
        for b in range(_NBUF):
            i = g * _NBUF + b
            s_wait(b)

            @pl.when(i + _NBUF < _NCHUNK)
            def _():
                idb_start(i + _NBUF, b)
                isb_wait(i + _NBUF, b)
                g_start(b)
        return carry

    lax.fori_loop(0, _NG, group, 0)
    for j in range(_NTAIL):
        i = _NG * _NBUF + j
        g_wait(j)
        idb_wait(i, j)
        s_start(j)
        s_wait(j)
    plsc.subcore_barrier()

    @pl.when(s < _NS - 1)
    def _():
        pltpu.sync_copy(agg_sh.at[pl.ds(row0, _SPAN)],
                        agg_hbm.at[c, pl.ds(row0, _SPAN)])

    @pl.when(s == _NS - 1)
    def _():
        pltpu.sync_copy(agg_sh.at[pl.ds(row0, _SPANL)],
                        agg_hbm.at[c, pl.ds(row0, _SPANL)])

    if with_deg:
        @pl.when(jnp.logical_and(s == 0, c == 0))
        def _():
            pltpu.sync_copy(deg_sh, deg0_hbm)

        @pl.when(jnp.logical_and(s == 0, c == 1))
        def _():
            pltpu.sync_copy(deg_sh, deg1_hbm)


def _make_sc_agg(with_deg, interpret=False):
    out_type = [jax.ShapeDtypeStruct((_NC, N_NODES, D), jnp.float32)]
    scratch = [
        pltpu.VMEM((_NBUF, _K), jnp.int32),
        pltpu.VMEM((_NBUF, _K), jnp.int32),
        pltpu.VMEM((_NBUF, _K, D), jnp.float32),
    ]
    if with_deg:
        out_type += [jax.ShapeDtypeStruct((N_NODES,), jnp.float32),
                     jax.ShapeDtypeStruct((N_NODES,), jnp.float32)]
        scratch += [
            pltpu.VMEM((_K + 16,), jnp.float32),
            pltpu.VMEM((_ZED,), jnp.float32),
        ]
    scratch.append(pltpu.VMEM_SHARED((N_NODES, D), jnp.float32))
    if with_deg:
        scratch.append(pltpu.VMEM_SHARED((N_NODES,), jnp.float32))
    nsem = 5 * _NBUF if with_deg else 4 * _NBUF
    scratch += [pltpu.SemaphoreType.DMA] * nsem
    mesh = plsc.VectorSubcoreMesh(core_axis_name="c", subcore_axis_name="s")
    return pl.kernel(
        functools.partial(_sc_body, with_deg),
        out_type=tuple(out_type) if with_deg else out_type[0],
        mesh=mesh,
        scratch_types=scratch,
        interpret=interpret,
        name="sage_sc_agg_deg" if with_deg else "sage_sc_agg",
    )


_R = 1000


def _tc_body(relu, h_ref, a0_ref, a1_ref, d0_ref, d1_ref, ws_ref, wn_ref, b_ref,
             o_ref):
    deg = jnp.maximum(d0_ref[...] + d1_ref[...], 1.0)
    neigh = (a0_ref[...] + a1_ref[...]) / deg
    y = jnp.dot(h_ref[...], ws_ref[...], preferred_element_type=jnp.float32)
    y += jnp.dot(neigh, wn_ref[...], preferred_element_type=jnp.float32)
    y += b_ref[...]
    if relu:
        y = jnp.maximum(y, 0.0)
    o_ref[...] = y


def _make_tc_layer(relu):
    blk = lambda i: (i, 0)
    full = lambda i: (0, 0)
    return pl.pallas_call(
        functools.partial(_tc_body, relu),
        grid=(N_NODES // _R,),
        in_specs=[
            pl.BlockSpec((_R, D), blk),
            pl.BlockSpec((_R, D), blk),
            pl.BlockSpec((_R, D), blk),
            pl.BlockSpec((_R, 1), blk),
            pl.BlockSpec((_R, 1), blk),
            pl.BlockSpec((D, D), full),
            pl.BlockSpec((D, D), full),
            pl.BlockSpec((1, D), full),
        ],
        out_specs=pl.BlockSpec((_R, D), blk),
        out_shape=jax.ShapeDtypeStruct((N_NODES, D), jnp.float32),
        name="sage_tc_layer",
    )


_sc_agg_deg = _make_sc_agg(with_deg=True)
_sc_agg = _make_sc_agg(with_deg=False)
_tc_layer_relu = _make_tc_layer(relu=True)
_tc_layer_lin = _make_tc_layer(relu=False)


def kernel(x, edge_index, W_self0, W_neigh0, b0, W_self1, W_neigh1, b1):
    src = edge_index[0].astype(jnp.int32)
    dst = edge_index[1].astype(jnp.int32)
    aggp, deg0, deg1 = _sc_agg_deg(x, src, dst)
    d0 = deg0.reshape(N_NODES, 1)
    d1 = deg1.reshape(N_NODES, 1)
    h = _tc_layer_relu(x, aggp[0], aggp[1], d0, d1, W_self0, W_neigh0,
                       b0.reshape(1, D))
    aggp2 = _sc_agg(h, src, dst)
    return _tc_layer_lin(h, aggp2[0], aggp2[1], d0, d1, W_self1, W_neigh1,
                         b1.reshape(1, D))


# speedup vs baseline: 12.8610x; 1.0462x over previous
import functools

import jax
import jax.numpy as jnp
from jax import lax
from jax.experimental import pallas as pl
from jax.experimental.pallas import tpu as pltpu
from jax.experimental.pallas import tpu_sc as plsc

N_NODES = 10000
N_EDGES = 320000
D = 128

_NC = 2
_NS = 16
_NW = _NC * _NS
_EPW = N_EDGES // _NW
_K = 80
_NCHUNK = _EPW // _K
_NBUF = 4
_NG = _NCHUNK // _NBUF
_NTAIL = _NCHUNK - _NG * _NBUF
_ZROWS = 8
_ZED = 1000
_SPAN = 632
_SPANL = N_NODES - (_NS - 1) * _SPAN


def _sc_body(with_deg, h_hbm, src_hbm, dst_hbm, agg_hbm, *rest):
    if with_deg:
        deg0_hbm, deg1_hbm = rest[0], rest[1]
        rest = rest[2:]
        (sbuf, dbuf, rows_v, ones_v, zed_v, agg_sh, deg_sh) = rest[:7]
        sems = rest[7:]
    else:
        (sbuf, dbuf, rows_v, agg_sh) = rest[:4]
        sems = rest[4:]
    gsem, ssem = sems[:_NBUF], sems[_NBUF:2 * _NBUF]
    issem, idsem = sems[2 * _NBUF:3 * _NBUF], sems[3 * _NBUF:4 * _NBUF]
    if with_deg:
        dsem = sems[4 * _NBUF:]
    c = lax.axis_index("c")
    s = lax.axis_index("s")
    wid = c * _NS + s

    def _ioff(i):
        return pl.multiple_of(wid * _EPW + i * _K, 16)

    def isb_start(i, b):
        pltpu.async_copy(src_hbm.at[pl.ds(_ioff(i), _K)], sbuf.at[b], issem[b])

    def isb_wait(i, b):
        pltpu.make_async_copy(src_hbm.at[pl.ds(_ioff(i), _K)], sbuf.at[b],
                              issem[b]).wait()

    def idb_start(i, b):
        pltpu.async_copy(dst_hbm.at[pl.ds(_ioff(i), _K)], dbuf.at[b], idsem[b])

    def idb_wait(i, b):
        pltpu.make_async_copy(dst_hbm.at[pl.ds(_ioff(i), _K)], dbuf.at[b],
                              idsem[b]).wait()

    for b in range(_NBUF):
        isb_start(b, b)
        idb_start(b, b)

    zvec = jnp.zeros((16,), jnp.float32)
    for r in range(_ZROWS):
        for j in range(D // 16):
            rows_v[0, r, pl.ds(16 * j, 16)] = zvec
    zstage = rows_v.at[0].at[pl.ds(0, _ZROWS)]
    row0 = pl.multiple_of(s * _SPAN, 8)
    span = jnp.where(s < _NS - 1, _SPAN, _SPANL)

    def zero_rows(i, carry):
        pltpu.sync_copy(zstage,
                        agg_sh.at[pl.ds(pl.multiple_of(row0 + i * _ZROWS, 8),
                                        _ZROWS)])
        return carry

    lax.fori_loop(0, span // _ZROWS, zero_rows, 0)

    if with_deg:
        for j in range(_K // 16 + 1):
            ones_v[pl.ds(16 * j, 16)] = jnp.ones((16,), jnp.float32)

        @pl.when(s == 0)
        def _():
            def zfill(i, carry):
                zed_v[pl.ds(i * 16, 16)] = zvec
                return carry

            lax.fori_loop(0, _ZED // 16, zfill, 0)

            def zdeg(i, carry):
                pltpu.sync_copy(zed_v,
                                deg_sh.at[pl.ds(pl.multiple_of(i * _ZED, 8),
                                                _ZED)])
                return carry

            lax.fori_loop(0, N_NODES // _ZED, zdeg, 0)

    plsc.subcore_barrier()

    def g_start(b):
        pltpu.async_copy(h_hbm.at[sbuf.at[b]], rows_v.at[b], gsem[b])

    def g_wait(b):
        pltpu.make_async_copy(h_hbm.at[sbuf.at[b]], rows_v.at[b],
                              gsem[b]).wait()

    def s_start(b):
        pltpu.async_copy(rows_v.at[b], agg_sh.at[dbuf.at[b]], ssem[b],
                         add=True)
        if with_deg:
            pltpu.async_copy(ones_v.at[pl.ds(0, _K)], deg_sh.at[dbuf.at[b]],
                             dsem[b], add=True)

    def s_wait(b):
        pltpu.make_async_copy(rows_v.at[b], agg_sh.at[dbuf.at[b]],
                              ssem[b]).wait()
        if with_deg:
            pltpu.make_async_copy(ones_v.at[pl.ds(0, _K)],
                                  deg_sh.at[dbuf.at[b]], dsem[b]).wait()

    for b in range(_NBUF):
        isb_wait(b, b)
        g_start(b)

    def group(g, carry):
        for b in range(_NBUF):
            i = g * _NBUF + b
            g_wait(b)

            @pl.when(i + _NBUF < _NCHUNK)
            def _():
                isb_start(i + _NBUF, b)

            idb_wait(i, b)
            s_start(b)
        for b in range(_NBUF):
            i = g * _NBUF + b
            s_wait(b)

            @pl.when(i + _NBUF < _NCHUNK)
            def _():
                idb_start(i + _NBUF, b)
                isb_wait(i + _NBUF, b)
                g_start(b)
        return carry

    lax.fori_loop(0, _NG, group, 0)
    for j in range(_NTAIL):
        i = _NG * _NBUF + j
        g_wait(j)
        idb_wait(i, j)
        s_start(j)
        s_wait(j)
    plsc.subcore_barrier()

    @pl.when(s < _NS - 1)
    def _():
        pltpu.sync_copy(agg_sh.at[pl.ds(row0, _SPAN)],
                        agg_hbm.at[c, pl.ds(row0, _SPAN)])

    @pl.when(s == _NS - 1)
    def _():
        pltpu.sync_copy(agg_sh.at[pl.ds(row0, _SPANL)],
                        agg_hbm.at[c, pl.ds(row0, _SPANL)])

    if with_deg:
        @pl.when(jnp.logical_and(s == 0, c == 0))
        def _():
            pltpu.sync_copy(deg_sh, deg0_hbm)

        @pl.when(jnp.logical_and(s == 0, c == 1))
        def _():
            pltpu.sync_copy(deg_sh, deg1_hbm)


def _make_sc_agg(with_deg, interpret=False):
    out_type = [jax.ShapeDtypeStruct((_NC, N_NODES, D), jnp.float32)]
    scratch = [
        pltpu.VMEM((_NBUF, _K), jnp.int32),
        pltpu.VMEM((_NBUF, _K), jnp.int32),
        pltpu.VMEM((_NBUF, _K, D), jnp.float32),
    ]
    if with_deg:
        out_type += [jax.ShapeDtypeStruct((N_NODES,), jnp.float32),
                     jax.ShapeDtypeStruct((N_NODES,), jnp.float32)]
        scratch += [
            pltpu.VMEM((_K + 16,), jnp.float32),
            pltpu.VMEM((_ZED,), jnp.float32),
        ]
    scratch.append(pltpu.VMEM_SHARED((N_NODES, D), jnp.float32))
    if with_deg:
        scratch.append(pltpu.VMEM_SHARED((N_NODES,), jnp.float32))
    nsem = 5 * _NBUF if with_deg else 4 * _NBUF
    scratch += [pltpu.SemaphoreType.DMA] * nsem
    mesh = plsc.VectorSubcoreMesh(core_axis_name="c", subcore_axis_name="s")
    return pl.kernel(
        functools.partial(_sc_body, with_deg),
        out_type=tuple(out_type) if with_deg else out_type[0],
        mesh=mesh,
        scratch_types=scratch,
        interpret=interpret,
        name="sage_sc_agg_deg" if with_deg else "sage_sc_agg",
    )


_R = 1000


def _tc_body(relu, h_ref, a_ref, d0_ref, d1_ref, ws_ref, wn_ref, b_ref,
             o_ref):
    deg = jnp.maximum(d0_ref[...] + d1_ref[...], 1.0)
    neigh = (a_ref[0] + a_ref[1]) / deg
    y = jnp.dot(h_ref[...], ws_ref[...], preferred_element_type=jnp.float32)
    y += jnp.dot(neigh, wn_ref[...], preferred_element_type=jnp.float32)
    y += b_ref[...]
    if relu:
        y = jnp.maximum(y, 0.0)
    o_ref[...] = y


def _make_tc_layer(relu):
    blk = lambda i: (i, 0)
    full = lambda i: (0, 0)
    return pl.pallas_call(
        functools.partial(_tc_body, relu),
        grid=(N_NODES // _R,),
        in_specs=[
            pl.BlockSpec((_R, D), blk),
            pl.BlockSpec((2, _R, D), lambda i: (0, i, 0)),
            pl.BlockSpec((_R, 1), blk),
            pl.BlockSpec((_R, 1), blk),
            pl.BlockSpec((D, D), full),
            pl.BlockSpec((D, D), full),
            pl.BlockSpec((1, D), full),
        ],
        out_specs=pl.BlockSpec((_R, D), blk),
        out_shape=jax.ShapeDtypeStruct((N_NODES, D), jnp.float32),
        name="sage_tc_layer",
    )


_sc_agg_deg = _make_sc_agg(with_deg=True)
_sc_agg = _make_sc_agg(with_deg=False)
_tc_layer_relu = _make_tc_layer(relu=True)
_tc_layer_lin = _make_tc_layer(relu=False)


def kernel(x, edge_index, W_self0, W_neigh0, b0, W_self1, W_neigh1, b1):
    src = edge_index[0].astype(jnp.int32)
    dst = edge_index[1].astype(jnp.int32)
    aggp, deg0, deg1 = _sc_agg_deg(x, src, dst)
    d0 = deg0.reshape(N_NODES, 1)
    d1 = deg1.reshape(N_NODES, 1)
    h = _tc_layer_relu(x, aggp, d0, d1, W_self0, W_neigh0, b0.reshape(1, D))
    aggp2 = _sc_agg(h, src, dst)
    return _tc_layer_lin(h, aggp2, d0, d1, W_self1, W_neigh1, b1.reshape(1, D))
